# baseline (speedup 1.0000x reference)
import functools

import jax
import jax.numpy as jnp
from jax import lax
from jax.experimental import pallas as pl
from jax.experimental.pallas import tpu as pltpu
from jax.experimental.pallas import tpu_sc as plsc

GH, GW, GD = 80, 80, 6
C = 18
CF = 17
N = 2048
LOX, LOY, LOZ = -40.0, -40.0, -1.0
NC, NS, L = 2, 16, 16
TI, TJ = 20, 10
TPI, TPJ = GH // TI, GW // TJ
NG1 = N // L

OFF_MEAN = 0
OFF_SCALE = OFF_MEAN + 3 * N
OFF_ROT = OFF_SCALE + 3 * N
OFF_OPA = OFF_ROT + 4 * N
OFF_FEAT = OFF_OPA + N
OFF_ES = OFF_FEAT + CF * N
PACKED = OFF_ES + L

_BGX = -0.5 / (80.0 * 80.0)
_BGZ = -0.5 / (6.4 * 6.4)
_CZ = 2.2


def _sc_body(packed, out, vin, va, vb, vc, vd, ve, vf,
             vi0, vi1, vj0, vj1, vk0, vk1, vlist, vslab, dsem):
    f32, i32 = jnp.float32, jnp.int32
    cid = lax.axis_index("c")
    sid = lax.axis_index("s")
    wid = sid * NC + cid
    tpi = wid // TPJ
    tpj = wid - tpi * TPJ
    ti0 = tpi * TI
    ti1 = ti0 + TI - 1
    tj0 = tpj * TJ
    tj1 = tj0 + TJ - 1

    pltpu.async_copy(packed, vin, dsem).wait()

    iota = lax.iota(i32, L)

    def p1(gi, cnt):
        gidx = gi * L + iota
        g3 = gidx * 3
        g4 = gidx * 4
        mxv = plsc.load_gather(vin, [g3 + OFF_MEAN])
        myv = plsc.load_gather(vin, [g3 + (OFF_MEAN + 1)])
        mzv = plsc.load_gather(vin, [g3 + (OFF_MEAN + 2)])
        sxv = plsc.load_gather(vin, [g3 + OFF_SCALE])
        syv = plsc.load_gather(vin, [g3 + (OFF_SCALE + 1)])
        szv = plsc.load_gather(vin, [g3 + (OFF_SCALE + 2)])
        qwv = plsc.load_gather(vin, [g4 + OFF_ROT])
        qxv = plsc.load_gather(vin, [g4 + (OFF_ROT + 1)])
        qyv = plsc.load_gather(vin, [g4 + (OFF_ROT + 2)])
        qzv = plsc.load_gather(vin, [g4 + (OFF_ROT + 3)])

        xx = qxv * qxv; yy = qyv * qyv; zz = qzv * qzv
        xy = qxv * qyv; xz = qxv * qzv; yz = qyv * qzv
        wx = qwv * qxv; wy = qwv * qyv; wz = qwv * qzv
        r00 = 1.0 - 2.0 * (yy + zz); r01 = 2.0 * (xy - wz); r02 = 2.0 * (xz + wy)
        r10 = 2.0 * (xy + wz); r11 = 1.0 - 2.0 * (xx + zz); r12 = 2.0 * (yz - wx)
        r20 = 2.0 * (xz - wy); r21 = 2.0 * (yz + wx); r22 = 1.0 - 2.0 * (xx + yy)
        e0 = 1.0 / (sxv * sxv); e1 = 1.0 / (syv * syv); e2 = 1.0 / (szv * szv)
        plsc.store_scatter(va, [gidx], -0.5 * (r00 * r00 * e0 + r01 * r01 * e1 + r02 * r02 * e2))
        plsc.store_scatter(vb, [gidx], -0.5 * (r10 * r10 * e0 + r11 * r11 * e1 + r12 * r12 * e2))
        plsc.store_scatter(vc, [gidx], -0.5 * (r20 * r20 * e0 + r21 * r21 * e1 + r22 * r22 * e2))
        plsc.store_scatter(vd, [gidx], -(r00 * r10 * e0 + r01 * r11 * e1 + r02 * r12 * e2))
        plsc.store_scatter(ve, [gidx], -(r00 * r20 * e0 + r01 * r21 * e1 + r02 * r22 * e2))
        plsc.store_scatter(vf, [gidx], -(r10 * r20 * e0 + r11 * r21 * e1 + r12 * r22 * e2))

        def lohi(m, s, lo, imax):
            tlo = m - 3.0 * s - (lo + 0.5)
            thi = m + 3.0 * s - (lo + 0.5)
            t0 = jnp.maximum(tlo, 0.0)
            c0 = t0.astype(i32)
            lo_i = c0 + (c0.astype(f32) < t0).astype(i32)
            t1 = jnp.minimum(thi, float(imax))
            c1 = t1.astype(i32)
            hi_i = c1 - (c1.astype(f32) > t1).astype(i32)
            return lo_i, hi_i

        i0v, i1v = lohi(mxv, sxv, LOX, GH - 1)
        j0v, j1v = lohi(myv, syv, LOY, GW - 1)
        k0v, k1v = lohi(mzv, szv, LOZ, GD - 1)
        plsc.store_scatter(vi0, [gidx], i0v)
        plsc.store_scatter(vi1, [gidx], i1v)
        plsc.store_scatter(vj0, [gidx], j0v)
        plsc.store_scatter(vj1, [gidx], j1v)
        plsc.store_scatter(vk0, [gidx], k0v)
        plsc.store_scatter(vk1, [gidx], k1v)

        inter = ((i0v <= ti1) & (i1v >= ti0) & (j0v <= tj1) & (j1v >= tj0)
                 & (i0v <= i1v) & (j0v <= j1v) & (k0v <= k1v))
        csum = plsc.cumsum(inter.astype(i32))
        pos = cnt + csum - 1
        plsc.store_scatter(vlist, [pos], gidx, mask=inter)
        return cnt + jnp.max(csum)

    count = lax.fori_loop(0, NG1, p1, jnp.int32(0))

    zeros = jnp.zeros((L,), f32)
    ch17 = jnp.full((L,), C - 1, dtype=i32)

    def pz(z, carry):
        lidx = z * L + iota
        vox = lidx // C
        ch = lidx - vox * C
        li = vox // (TJ * GD)
        r = vox - li * (TJ * GD)
        lj = r // GD
        k = r - lj * GD
        plsc.store_scatter(vslab, [li, lj, k, ch], zeros)
        return carry

    lax.fori_loop(0, (TI * TJ * GD * C) // L, pz, jnp.int32(0))

    esv = vin[OFF_ES:OFF_ES + L]

    def pb(v, carry):
        lidx = v * L + iota
        li = lidx // (TJ * GD)
        r = lidx - li * (TJ * GD)
        lj = r // GD
        k = r - lj * GD
        dx = (ti0 + li).astype(f32) + (LOX + 0.5)
        dy = (tj0 + lj).astype(f32) + (LOY + 0.5)
        dz = k.astype(f32) + (LOZ + 0.5 - _CZ)
        w = esv * jnp.exp(dx * dx * _BGX + dy * dy * _BGX + dz * dz * _BGZ)
        plsc.store_scatter(vslab, [li, lj, k, ch17], w)
        return carry

    lax.fori_loop(0, (TI * TJ * GD) // L, pb, jnp.int32(0))

    def p2(t, carry):
        tvec = jnp.full((L,), t, dtype=i32)
        g = plsc.load_gather(vlist, [tvec])
        g3 = g * 3
        mxg = plsc.load_gather(vin, [g3 + OFF_MEAN])
        myg = plsc.load_gather(vin, [g3 + (OFF_MEAN + 1)])
        mzg = plsc.load_gather(vin, [g3 + (OFF_MEAN + 2)])
        ag = plsc.load_gather(va, [g])
        bg = plsc.load_gather(vb, [g])
        cg = plsc.load_gather(vc, [g])
        dg = plsc.load_gather(vd, [g])
        eg = plsc.load_gather(ve, [g])
        fg = plsc.load_gather(vf, [g])
        og = plsc.load_gather(vin, [g + OFF_OPA])
        i0g = plsc.load_gather(vi0, [g])
        i1g = plsc.load_gather(vi1, [g])
        j0g = plsc.load_gather(vj0, [g])
        j1g = plsc.load_gather(vj1, [g])
        k0g = plsc.load_gather(vk0, [g])
        k1g = plsc.load_gather(vk1, [g])
        g17 = g * CF
        fcs = [plsc.load_gather(vin, [g17 + (OFF_FEAT + ch)]) for ch in range(CF)]

        ii0 = jnp.maximum(i0g, ti0)
        ii1 = jnp.minimum(i1g, ti1)
        jj0 = jnp.maximum(j0g, tj0)
        jj1 = jnp.minimum(j1g, tj1)
        njv = jj1 - jj0 + 1
        nkv = k1g - k0g + 1
        njk = njv * nkv
        nvox = (ii1 - ii0 + 1) * njk
        rjk = 1.0 / njk.astype(f32)
        rk = 1.0 / nkv.astype(f32)
        ng = (jnp.max(nvox) + (L - 1)) // L

        def inner(u, c2):
            l = u * L + iota
            lf = l.astype(f32) + 0.5
            di = (lf * rjk).astype(i32)
            r = l - di * njk
            dj = ((r.astype(f32) + 0.5) * rk).astype(i32)
            dk = r - dj * nkv
            valid = l < nvox
            i = ii0 + di
            j = jj0 + dj
            k = k0g + dk
            dx = i.astype(f32) + (LOX + 0.5) - mxg
            dy = j.astype(f32) + (LOY + 0.5) - myg
            dz = k.astype(f32) + (LOZ + 0.5) - mzg
            q = (ag * dx * dx + bg * dy * dy + cg * dz * dz
                 + dg * dx * dy + eg * dx * dz + fg * dy * dz)
            w = og * jnp.exp(q)
            li = jnp.where(valid, i - ti0, 0)
            lj = jnp.where(valid, j - tj0, 0)
            k = jnp.where(valid, k, 0)
            for ch in range(CF):
                chv = jnp.full((L,), ch, dtype=i32)
                plsc.addupdate_scatter(vslab, [li, lj, k, chv], w * fcs[ch],
                                       mask=valid)
            return c2

        lax.fori_loop(0, ng, inner, jnp.int32(0))
        return carry

    lax.fori_loop(0, count, p2, jnp.int32(0))

    odescs = []
    for li in range(TI):
        odescs.append(pltpu.async_copy(vslab.at[li],
                                       out.at[ti0 + li, pl.ds(tj0, TJ)], dsem))
    for d in odescs:
        d.wait()


@functools.lru_cache(maxsize=1)
def _build():
    f32, i32 = jnp.float32, jnp.int32
    mesh = plsc.VectorSubcoreMesh(core_axis_name="c", subcore_axis_name="s",
                                  num_cores=NC, num_subcores=NS)
    scratch = (
        [pltpu.VMEM((PACKED,), f32)]
        + [pltpu.VMEM((N,), f32) for _ in range(6)]
        + [pltpu.VMEM((N,), i32) for _ in range(6)]
        + [pltpu.VMEM((N,), i32)]
        + [pltpu.VMEM((TI, TJ, GD, C), f32),
           pltpu.SemaphoreType.DMA]
    )
    return pl.kernel(
        _sc_body,
        out_type=jax.ShapeDtypeStruct((GH, GW, GD, C), f32),
        mesh=mesh,
        scratch_types=scratch,
        compiler_params=pltpu.CompilerParams(needs_layout_passes=False,
                                             use_tc_tiling_on_sc=False),
    )


def kernel(means3d, opacities, scales, rotations, features, empty_scalar):
    f32 = jnp.float32
    packed = jnp.concatenate([
        means3d.astype(f32).reshape(-1), scales.astype(f32).reshape(-1),
        rotations.astype(f32).reshape(-1), opacities.astype(f32).reshape(-1),
        features.astype(f32).reshape(-1),
        jnp.broadcast_to(empty_scalar.astype(f32).reshape(-1)[:1], (L,)),
    ])
    grid_feats = _build()(packed)
    grid_density = jnp.zeros((GH, GW, GD, 1), f32)
    return grid_density, grid_feats


# speedup vs baseline: 2.6505x; 2.6505x over previous
import functools

import jax
import jax.numpy as jnp
from jax import lax
from jax.experimental import pallas as pl
from jax.experimental.pallas import tpu as pltpu
from jax.experimental.pallas import tpu_sc as plsc

GH, GW, GD = 80, 80, 6
C = 18
CF = 17
N = 2048
LOX, LOY, LOZ = -40.0, -40.0, -1.0
NC, NS, L = 2, 16, 16
TI, TJ = 20, 10
TPI, TPJ = GH // TI, GW // TJ
NG1 = N // L

OFF_MEAN = 0
OFF_SCALE = OFF_MEAN + 3 * N
OFF_ROT = OFF_SCALE + 3 * N
OFF_OPA = OFF_ROT + 4 * N
OFF_FEAT = OFF_OPA + N
OFF_ES = OFF_FEAT + CF * N
PACKED = OFF_ES + L

_BGX = -0.5 / (80.0 * 80.0)
_BGZ = -0.5 / (6.4 * 6.4)
_CZ = 2.2


def _sc_body(packed, out, vin, va, vb, vc, vd, ve, vf,
             vi0, vi1, vj0, vj1, vk0, vk1, vlist, vslab, dsem):
    f32, i32 = jnp.float32, jnp.int32
    cid = lax.axis_index("c")
    sid = lax.axis_index("s")
    wid = sid * NC + cid
    tpi = wid // TPJ
    tpj = wid - tpi * TPJ
    ti0 = tpi * TI
    ti1 = ti0 + TI - 1
    tj0 = tpj * TJ
    tj1 = tj0 + TJ - 1

    pltpu.async_copy(packed, vin, dsem).wait()

    iota = lax.iota(i32, L)

    def p1(gi, cnt):
        gidx = gi * L + iota
        g3 = gidx * 3
        g4 = gidx * 4
        mxv = plsc.load_gather(vin, [g3 + OFF_MEAN])
        myv = plsc.load_gather(vin, [g3 + (OFF_MEAN + 1)])
        mzv = plsc.load_gather(vin, [g3 + (OFF_MEAN + 2)])
        sxv = plsc.load_gather(vin, [g3 + OFF_SCALE])
        syv = plsc.load_gather(vin, [g3 + (OFF_SCALE + 1)])
        szv = plsc.load_gather(vin, [g3 + (OFF_SCALE + 2)])
        qwv = plsc.load_gather(vin, [g4 + OFF_ROT])
        qxv = plsc.load_gather(vin, [g4 + (OFF_ROT + 1)])
        qyv = plsc.load_gather(vin, [g4 + (OFF_ROT + 2)])
        qzv = plsc.load_gather(vin, [g4 + (OFF_ROT + 3)])

        xx = qxv * qxv; yy = qyv * qyv; zz = qzv * qzv
        xy = qxv * qyv; xz = qxv * qzv; yz = qyv * qzv
        wx = qwv * qxv; wy = qwv * qyv; wz = qwv * qzv
        r00 = 1.0 - 2.0 * (yy + zz); r01 = 2.0 * (xy - wz); r02 = 2.0 * (xz + wy)
        r10 = 2.0 * (xy + wz); r11 = 1.0 - 2.0 * (xx + zz); r12 = 2.0 * (yz - wx)
        r20 = 2.0 * (xz - wy); r21 = 2.0 * (yz + wx); r22 = 1.0 - 2.0 * (xx + yy)
        e0 = 1.0 / (sxv * sxv); e1 = 1.0 / (syv * syv); e2 = 1.0 / (szv * szv)
        plsc.store_scatter(va, [gidx], -0.5 * (r00 * r00 * e0 + r01 * r01 * e1 + r02 * r02 * e2))
        plsc.store_scatter(vb, [gidx], -0.5 * (r10 * r10 * e0 + r11 * r11 * e1 + r12 * r12 * e2))
        plsc.store_scatter(vc, [gidx], -0.5 * (r20 * r20 * e0 + r21 * r21 * e1 + r22 * r22 * e2))
        plsc.store_scatter(vd, [gidx], -(r00 * r10 * e0 + r01 * r11 * e1 + r02 * r12 * e2))
        plsc.store_scatter(ve, [gidx], -(r00 * r20 * e0 + r01 * r21 * e1 + r02 * r22 * e2))
        plsc.store_scatter(vf, [gidx], -(r10 * r20 * e0 + r11 * r21 * e1 + r12 * r22 * e2))

        def lohi(m, s, lo, imax):
            tlo = m - 3.0 * s - (lo + 0.5)
            thi = m + 3.0 * s - (lo + 0.5)
            t0 = jnp.maximum(tlo, 0.0)
            c0 = t0.astype(i32)
            lo_i = c0 + (c0.astype(f32) < t0).astype(i32)
            t1 = jnp.minimum(thi, float(imax))
            c1 = t1.astype(i32)
            hi_i = c1 - (c1.astype(f32) > t1).astype(i32)
            return lo_i, hi_i

        i0v, i1v = lohi(mxv, sxv, LOX, GH - 1)
        j0v, j1v = lohi(myv, syv, LOY, GW - 1)
        k0v, k1v = lohi(mzv, szv, LOZ, GD - 1)
        plsc.store_scatter(vi0, [gidx], i0v)
        plsc.store_scatter(vi1, [gidx], i1v)
        plsc.store_scatter(vj0, [gidx], j0v)
        plsc.store_scatter(vj1, [gidx], j1v)
        plsc.store_scatter(vk0, [gidx], k0v)
        plsc.store_scatter(vk1, [gidx], k1v)

        inter = ((i0v <= ti1) & (i1v >= ti0) & (j0v <= tj1) & (j1v >= tj0)
                 & (i0v <= i1v) & (j0v <= j1v) & (k0v <= k1v))
        csum = plsc.cumsum(inter.astype(i32))
        pos = cnt + csum - 1
        plsc.store_scatter(vlist, [pos], gidx, mask=inter)
        return cnt + jnp.max(csum)

    count = lax.fori_loop(0, NG1, p1, jnp.int32(0))

    zeros = jnp.zeros((L,), f32)

    def pz(z, carry):
        lidx = z * L + iota
        col = lidx // (GD * C)
        inner = lidx - col * (GD * C)
        plsc.store_scatter(vslab, [col, inner], zeros)
        return carry

    lax.fori_loop(0, (TI * TJ * GD * C) // L, pz, jnp.int32(0))

    esv = vin[OFF_ES:OFF_ES + L]

    def pb(v, carry):
        lidx = v * L + iota
        li = lidx // (TJ * GD)
        r = lidx - li * (TJ * GD)
        lj = r // GD
        k = r - lj * GD
        dx = (ti0 + li).astype(f32) + (LOX + 0.5)
        dy = (tj0 + lj).astype(f32) + (LOY + 0.5)
        dz = k.astype(f32) + (LOZ + 0.5 - _CZ)
        w = esv * jnp.exp(dx * dx * _BGX + dy * dy * _BGX + dz * dz * _BGZ)
        plsc.store_scatter(vslab, [li * TJ + lj, k * C + (C - 1)], w)
        return carry

    lax.fori_loop(0, (TI * TJ * GD) // L, pb, jnp.int32(0))

    def p2(t, carry):
        tvec = jnp.full((L,), t, dtype=i32)
        g = plsc.load_gather(vlist, [tvec])
        g3 = g * 3
        mxg = plsc.load_gather(vin, [g3 + OFF_MEAN])
        myg = plsc.load_gather(vin, [g3 + (OFF_MEAN + 1)])
        mzg = plsc.load_gather(vin, [g3 + (OFF_MEAN + 2)])
        ag = plsc.load_gather(va, [g])
        bg = plsc.load_gather(vb, [g])
        cg = plsc.load_gather(vc, [g])
        dg = plsc.load_gather(vd, [g])
        eg = plsc.load_gather(ve, [g])
        fg = plsc.load_gather(vf, [g])
        og = plsc.load_gather(vin, [g + OFF_OPA])
        i0g = plsc.load_gather(vi0, [g])
        i1g = plsc.load_gather(vi1, [g])
        j0g = plsc.load_gather(vj0, [g])
        j1g = plsc.load_gather(vj1, [g])
        k0g = plsc.load_gather(vk0, [g])
        k1g = plsc.load_gather(vk1, [g])
        g17 = g * CF
        fcs = [plsc.load_gather(vin, [g17 + (OFF_FEAT + ch)]) for ch in range(CF)]

        ii0 = jnp.maximum(i0g, ti0)
        ii1 = jnp.minimum(i1g, ti1)
        jj0 = jnp.maximum(j0g, tj0)
        jj1 = jnp.minimum(j1g, tj1)
        njv = jj1 - jj0 + 1
        nkv = k1g - k0g + 1
        njk = njv * nkv
        nvox = (ii1 - ii0 + 1) * njk
        rjk = 1.0 / njk.astype(f32)
        rk = 1.0 / nkv.astype(f32)
        ng = (jnp.max(nvox) + (L - 1)) // L

        def inner(u, c2):
            l = u * L + iota
            lf = l.astype(f32) + 0.5
            di = (lf * rjk).astype(i32)
            r = l - di * njk
            dj = ((r.astype(f32) + 0.5) * rk).astype(i32)
            dk = r - dj * nkv
            valid = l < nvox
            i = ii0 + di
            j = jj0 + dj
            k = k0g + dk
            dx = i.astype(f32) + (LOX + 0.5) - mxg
            dy = j.astype(f32) + (LOY + 0.5) - myg
            dz = k.astype(f32) + (LOZ + 0.5) - mzg
            q = (ag * dx * dx + bg * dy * dy + cg * dz * dz
                 + dg * dx * dy + eg * dx * dz + fg * dy * dz)
            w = og * jnp.exp(q)
            col = (i - ti0) * TJ + (j - tj0)
            inner = k * C
            col = jnp.where(valid, col, 0)
            inner = jnp.where(valid, inner, 0)
            for ch in range(CF):
                plsc.addupdate_scatter(vslab, [col, inner + ch], w * fcs[ch],
                                       mask=valid)
            return c2

        lax.fori_loop(0, ng, inner, jnp.int32(0))
        return carry

    lax.fori_loop(0, count, p2, jnp.int32(0))

    ROW = TJ * GD * C
    odescs = []
    for li in range(TI):
        odescs.append(pltpu.async_copy(vslab.at[pl.ds(li * TJ, TJ)],
                                       out.at[ti0 + li, pl.ds(tj0, TJ)], dsem))
    for d in odescs:
        d.wait()


@functools.lru_cache(maxsize=1)
def _build():
    f32, i32 = jnp.float32, jnp.int32
    mesh = plsc.VectorSubcoreMesh(core_axis_name="c", subcore_axis_name="s",
                                  num_cores=NC, num_subcores=NS)
    scratch = (
        [pltpu.VMEM((PACKED,), f32)]
        + [pltpu.VMEM((N,), f32) for _ in range(6)]
        + [pltpu.VMEM((N,), i32) for _ in range(6)]
        + [pltpu.VMEM((N,), i32)]
        + [pltpu.VMEM((TI * TJ, GD * C), f32),
           pltpu.SemaphoreType.DMA]
    )
    return pl.kernel(
        _sc_body,
        out_type=jax.ShapeDtypeStruct((GH, GW, GD * C), f32),
        mesh=mesh,
        scratch_types=scratch,
        compiler_params=pltpu.CompilerParams(needs_layout_passes=False,
                                             use_tc_tiling_on_sc=False,
                                             disable_bounds_checks=True),
    )


def kernel(means3d, opacities, scales, rotations, features, empty_scalar):
    f32 = jnp.float32
    packed = jnp.concatenate([
        means3d.astype(f32).reshape(-1), scales.astype(f32).reshape(-1),
        rotations.astype(f32).reshape(-1), opacities.astype(f32).reshape(-1),
        features.astype(f32).reshape(-1),
        jnp.broadcast_to(empty_scalar.astype(f32).reshape(-1)[:1], (L,)),
    ])
    grid_feats = _build()(packed).reshape(GH, GW, GD, C)
    grid_density = jnp.zeros((GH, GW, GD, 1), f32)
    return grid_density, grid_feats
